# baseline (speedup 1.0000x reference)
import jax
import jax.numpy as jnp
from jax.experimental import pallas as pl

H = 128
TILE = 4000


def _cell_kernel(x_ref, h_ref, c_ref, wil_ref, wir_ref, wfl_ref, wfr_ref,
                 uiou_ref, biou_ref, ufw_ref, ufb_ref, out_ref):
    wiou = wil_ref[...] + wir_ref[...]
    wf = wfl_ref[...] + wfr_ref[...]
    uf_sum = ufw_ref[:, :H] + ufw_ref[:, H:]
    bf_sum = ufb_ref[0, :H] + ufb_ref[0, H:]

    x = x_ref[...]
    h0 = h_ref[:, 0, :]
    h1 = h_ref[:, 1, :]
    iou = jnp.dot(x, wiou, preferred_element_type=jnp.float32)
    iou += jnp.dot(h0, uiou_ref[:H], preferred_element_type=jnp.float32)
    iou += jnp.dot(h1, uiou_ref[H:], preferred_element_type=jnp.float32)
    iou += biou_ref[...]
    eq2 = jnp.dot(x, wf, preferred_element_type=jnp.float32)
    eq2 += jnp.dot(h0, uf_sum[:H], preferred_element_type=jnp.float32)
    eq2 += jnp.dot(h1, uf_sum[H:], preferred_element_type=jnp.float32)
    eq2 += bf_sum[None, :]

    i = jax.nn.sigmoid(iou[:, :H])
    o = jax.nn.sigmoid(iou[:, H:2 * H])
    u = jnp.tanh(iou[:, 2 * H:])
    f = jax.nn.sigmoid(eq2)
    c = i * u + f * (c_ref[:, 0, :] + c_ref[:, 1, :])
    out_ref[:, :H] = o * jnp.tanh(c)
    out_ref[:, H:] = c


def kernel(x, h_child, c_child, W_iou_left, W_iou_right, W_f_left, W_f_right,
           U_iou, b_iou, U_f_w, U_f_b):
    p = x.shape[0]
    rep = lambda *dims: pl.BlockSpec(dims, lambda i: (0,) * len(dims))
    grid = (p // TILE,)
    out = pl.pallas_call(
        _cell_kernel,
        grid=grid,
        in_specs=[
            pl.BlockSpec((TILE, H), lambda i: (i, 0)),
            pl.BlockSpec((TILE, 2, H), lambda i: (i, 0, 0)),
            pl.BlockSpec((TILE, 2, H), lambda i: (i, 0, 0)),
            rep(H, 3 * H),
            rep(H, 3 * H),
            rep(H, H),
            rep(H, H),
            rep(2 * H, 3 * H),
            rep(1, 3 * H),
            rep(2 * H, 2 * H),
            rep(1, 2 * H),
        ],
        out_specs=pl.BlockSpec((TILE, 2 * H), lambda i: (i, 0)),
        out_shape=jax.ShapeDtypeStruct((p, 2 * H), jnp.float32),
    )(x, h_child, c_child, W_iou_left, W_iou_right, W_f_left, W_f_right,
      U_iou, b_iou, U_f_w, U_f_b.reshape(1, 2 * H))
    return out


# speedup vs baseline: 1.3792x; 1.3792x over previous
import jax
import jax.numpy as jnp
from jax.experimental import pallas as pl
from jax.experimental.pallas import tpu as pltpu

H = 128
TILE = 5000


def _cell_kernel(x_ref, h_hbm, c_hbm, wx_ref, wh0_ref, wh1_ref, b_ref,
                 out_ref, h0_buf, h1_buf, c0_buf, c1_buf, sems):
    g = pl.program_id(0)
    n_tiles = pl.num_programs(0)
    bufs = (h0_buf, h1_buf, c0_buf, c1_buf)

    def copies(tile_idx, slot):
        base = tile_idx * TILE
        srcs = (h_hbm.at[pl.ds(base, TILE), 0, :],
                h_hbm.at[pl.ds(base, TILE), 1, :],
                c_hbm.at[pl.ds(base, TILE), 0, :],
                c_hbm.at[pl.ds(base, TILE), 1, :])
        return [pltpu.make_async_copy(src, buf.at[slot], sems.at[slot, j])
                for j, (src, buf) in enumerate(zip(srcs, bufs))]

    @pl.when(g == 0)
    def _():
        for cp in copies(0, 0):
            cp.start()

    @pl.when(g + 1 < n_tiles)
    def _():
        for cp in copies(g + 1, (g + 1) % 2):
            cp.start()

    slot = g % 2
    for cp in copies(g, slot):
        cp.wait()

    acc = jnp.dot(x_ref[...], wx_ref[...], preferred_element_type=jnp.float32)
    acc += jnp.dot(h0_buf[slot], wh0_ref[...], preferred_element_type=jnp.float32)
    acc += jnp.dot(h1_buf[slot], wh1_ref[...], preferred_element_type=jnp.float32)
    acc += b_ref[...]
    i = 0.5 * jnp.tanh(0.5 * acc[:, :H]) + 0.5
    o = 0.5 * jnp.tanh(0.5 * acc[:, H:2 * H]) + 0.5
    u = jnp.tanh(acc[:, 2 * H:3 * H])
    f = 0.5 * jnp.tanh(0.5 * acc[:, 3 * H:]) + 0.5
    c = i * u + f * (c0_buf[slot] + c1_buf[slot])
    out_ref[:, :H] = o * jnp.tanh(c)
    out_ref[:, H:] = c


def kernel(x, h_child, c_child, W_iou_left, W_iou_right, W_f_left, W_f_right,
           U_iou, b_iou, U_f_w, U_f_b):
    p = x.shape[0]
    wx = jnp.concatenate([W_iou_left + W_iou_right, W_f_left + W_f_right], axis=1)
    wh = jnp.concatenate([U_iou, U_f_w[:, :H] + U_f_w[:, H:]], axis=1)
    bias = jnp.concatenate([b_iou[0], U_f_b[:H] + U_f_b[H:]])[None, :]

    grid = (p // TILE,)
    vbuf = pltpu.VMEM((2, TILE, H), jnp.float32)
    out = pl.pallas_call(
        _cell_kernel,
        grid=grid,
        in_specs=[
            pl.BlockSpec((TILE, H), lambda i: (i, 0)),
            pl.BlockSpec(memory_space=pltpu.MemorySpace.HBM),
            pl.BlockSpec(memory_space=pltpu.MemorySpace.HBM),
            pl.BlockSpec((H, 4 * H), lambda i: (0, 0)),
            pl.BlockSpec((H, 4 * H), lambda i: (0, 0)),
            pl.BlockSpec((H, 4 * H), lambda i: (0, 0)),
            pl.BlockSpec((1, 4 * H), lambda i: (0, 0)),
        ],
        out_specs=pl.BlockSpec((TILE, 2 * H), lambda i: (i, 0)),
        out_shape=jax.ShapeDtypeStruct((p, 2 * H), jnp.float32),
        scratch_shapes=[vbuf, vbuf, vbuf, vbuf,
                        pltpu.SemaphoreType.DMA((2, 4))],
    )(x, h_child, c_child, wx, wh[:H], wh[H:], bias)
    return out
